# baseline (speedup 1.0000x reference)
import functools

import jax
import jax.numpy as jnp
from jax import lax
from jax.experimental import pallas as pl
from jax.experimental.pallas import tpu as pltpu
from jax.experimental.pallas import tpu_sc as plsc

EPS = 1e-5

_NC = 2
_NS = 16
_NW = _NC * _NS

_CB = 64

_NSEG = 1
_EB = 1280


def _sc_gather(xt, idx3):
    E, C = xt.shape
    EW = idx3.shape[2]
    NCH = EW // _CB
    EP = _NW * EW

    mesh = plsc.VectorSubcoreMesh(core_axis_name="c", subcore_axis_name="s")

    @functools.partial(
        pl.kernel,
        out_type=jax.ShapeDtypeStruct((4, EP, C), jnp.float32),
        mesh=mesh,
        scratch_types=[
            [pltpu.VMEM((EW,), jnp.int32) for _ in range(4)],
            pltpu.VMEM((4, _CB, C), jnp.float32),
            pltpu.VMEM((4, _CB, C), jnp.float32),
            pltpu.SemaphoreType.DMA,
            pltpu.SemaphoreType.DMA,
        ],
    )
    def gather_kernel(xt_hbm, idx3_hbm, g_hbm, idx_v, bufa, bufb, sema, semb):
        wid = lax.axis_index("s") * _NC + lax.axis_index("c")
        base = wid * EW
        for k in range(4):
            pltpu.sync_copy(idx3_hbm.at[wid, k], idx_v[k])

        def start(chunk, buf, sem):
            off = chunk * _CB
            for k in range(4):
                pltpu.async_copy(
                    xt_hbm.at[idx_v[k].at[pl.ds(off, _CB)]], buf.at[k], sem)

        def wait4(buf, sem):
            for k in range(4):
                pltpu.make_async_copy(
                    xt_hbm.at[pl.ds(0, _CB)], buf.at[k], sem).wait()

        def write(chunk, buf):
            off = base + chunk * _CB
            for k in range(4):
                pltpu.sync_copy(buf.at[k], g_hbm.at[k, pl.ds(off, _CB)])

        start(0, bufa, sema)

        def body(t, carry):
            i = t * 2
            wait4(bufa, sema)
            start(i + 1, bufb, semb)
            write(i, bufa)
            wait4(bufb, semb)
            start(i + 2, bufa, sema)
            write(i + 1, bufb)
            return carry

        lax.fori_loop(0, NCH // 2 - 1, body, 0)
        wait4(bufa, sema)
        start(NCH - 1, bufb, semb)
        write(NCH - 2, bufa)
        wait4(bufb, semb)
        write(NCH - 1, bufb)

    return gather_kernel(xt, idx3)


def _tc_conv(fe, g, wt, b2, seg, es):
    _, C, E = fe.shape
    c_out = wt.shape[1]
    nb = es // _EB
    off = seg * nb

    def body(fe_ref, g_ref, w_ref, b_ref, y_ref, st_ref, sacc, qacc):
        i = pl.program_id(0)

        @pl.when(i == 0)
        def _():
            sacc[...] = jnp.zeros_like(sacc)
            qacc[...] = jnp.zeros_like(qacc)

        xb = fe_ref[0]
        w = w_ref[...]
        gg = g_ref[...]
        s13 = gg[0] + gg[2]
        s24 = gg[1] + gg[3]
        d13 = jnp.abs(gg[0] - gg[2])
        d24 = jnp.abs(gg[1] - gg[3])
        dn = (((1,), (1,)), ((), ()))
        acc = jnp.dot(w[0], xb, preferred_element_type=jnp.float32)
        acc += lax.dot_general(w[1], s13, dn, preferred_element_type=jnp.float32)
        acc += lax.dot_general(w[2], s24, dn, preferred_element_type=jnp.float32)
        acc += lax.dot_general(w[3], d13, dn, preferred_element_type=jnp.float32)
        acc += lax.dot_general(w[4], d24, dn, preferred_element_type=jnp.float32)
        y = jnp.maximum(acc + b_ref[...], 0.0)
        y_ref[...] = y
        sacc[...] += jnp.sum(y, axis=1, keepdims=True)
        qacc[...] += jnp.sum(y * y, axis=1, keepdims=True)

        @pl.when(i == nb - 1)
        def _():
            st_ref[...] = jnp.concatenate([sacc[...], qacc[...]], axis=1)

    return pl.pallas_call(
        body,
        grid=(nb,),
        in_specs=[
            pl.BlockSpec((1, C, _EB), lambda i: (0, 0, i + off)),
            pl.BlockSpec((4, _EB, C), lambda i: (0, i, 0)),
            pl.BlockSpec((5, c_out, C), lambda i: (0, 0, 0)),
            pl.BlockSpec((c_out, 1), lambda i: (0, 0)),
        ],
        out_specs=[
            pl.BlockSpec((c_out, _EB), lambda i: (0, i)),
            pl.BlockSpec((c_out, 2), lambda i: (0, 0)),
        ],
        out_shape=[
            jax.ShapeDtypeStruct((c_out, es), jnp.float32),
            jax.ShapeDtypeStruct((c_out, 2), jnp.float32),
        ],
        scratch_shapes=[
            pltpu.VMEM((c_out, 1), jnp.float32),
            pltpu.VMEM((c_out, 1), jnp.float32),
        ],
        compiler_params=pltpu.CompilerParams(
            dimension_semantics=("arbitrary",)),
    )(fe, g, wt, b2)


def _tc_norm(y, st_all, n_total):
    c_out, es = y.shape
    nb = es // _EB
    nseg = st_all.shape[0]
    inv_e = float(1.0 / n_total)

    def body(y_ref, st_ref, o_ref):
        stv = jnp.sum(st_ref[...], axis=0)
        mu = stv[:, 0:1] * inv_e
        var = stv[:, 1:2] * inv_e - mu * mu
        r = lax.rsqrt(var + EPS)
        o_ref[...] = ((y_ref[...] - mu) * r)[None]

    return pl.pallas_call(
        body,
        grid=(nb,),
        in_specs=[
            pl.BlockSpec((c_out, _EB), lambda i: (0, i)),
            pl.BlockSpec((nseg, c_out, 2), lambda i: (0, 0, 0)),
        ],
        out_specs=pl.BlockSpec((1, c_out, _EB), lambda i: (0, 0, i)),
        out_shape=jax.ShapeDtypeStruct((1, c_out, es), jnp.float32),
        compiler_params=pltpu.CompilerParams(
            dimension_semantics=("arbitrary",)),
    )(y, st_all)


def kernel(fe, gemm_edges, W, b):
    _, C, E = fe.shape
    c_out = W.shape[0]
    es = E // _NSEG
    xt = jnp.transpose(fe[0])
    wt = jnp.transpose(W, (2, 0, 1))
    b2 = b.reshape(c_out, 1)
    ew = -(-es // (_NW * _CB)) * _CB
    if (ew // _CB) % 2:
        ew += _CB
    idx = gemm_edges[0]

    ys, sts = [], []
    for s in range(_NSEG):
        idx_s = idx[s * es:(s + 1) * es]
        if _NW * ew != es:
            idx_s = jnp.concatenate(
                [idx_s, jnp.zeros((_NW * ew - es, 4), dtype=idx.dtype)],
                axis=0)
        idx3 = jnp.transpose(idx_s.reshape(_NW, ew, 4), (0, 2, 1))
        g = _sc_gather(xt, idx3)
        y, st = _tc_conv(fe, g, wt, b2, s, es)
        ys.append(y)
        sts.append(st)

    st_all = jnp.stack(sts)
    outs = [_tc_norm(y, st_all, E) for y in ys]
    return jnp.concatenate(outs, axis=2)


# speedup vs baseline: 1.5942x; 1.5942x over previous
import functools

import jax
import jax.numpy as jnp
from jax import lax
from jax.experimental import pallas as pl
from jax.experimental.pallas import tpu as pltpu
from jax.experimental.pallas import tpu_sc as plsc

EPS = 1e-5

_NC = 2
_NS = 16
_NW = _NC * _NS

_CB = 40

_NSEG = 1
_EB = 1280


def _sc_gather(xt, idx3):
    E, C = xt.shape
    EW = idx3.shape[2]
    NCH = EW // _CB
    EP = _NW * EW

    mesh = plsc.VectorSubcoreMesh(core_axis_name="c", subcore_axis_name="s")

    @functools.partial(
        pl.kernel,
        out_type=jax.ShapeDtypeStruct((4, EP, C), jnp.float32),
        mesh=mesh,
        scratch_types=[
            [pltpu.VMEM((EW,), jnp.int32) for _ in range(4)],
            pltpu.VMEM((4, _CB, C), jnp.float32),
            pltpu.VMEM((4, _CB, C), jnp.float32),
            pltpu.SemaphoreType.DMA,
            pltpu.SemaphoreType.DMA,
        ],
    )
    def gather_kernel(xt_hbm, idx3_hbm, g_hbm, idx_v, bufa, bufb, sema, semb):
        wid = lax.axis_index("s") * _NC + lax.axis_index("c")
        base = wid * EW
        for k in range(4):
            pltpu.sync_copy(idx3_hbm.at[wid, k], idx_v[k])

        def start(chunk, buf, sem):
            off = chunk * _CB
            for k in range(4):
                pltpu.async_copy(
                    xt_hbm.at[idx_v[k].at[pl.ds(off, _CB)]], buf.at[k], sem)

        def wait4(buf, sem):
            for k in range(4):
                pltpu.make_async_copy(
                    xt_hbm.at[pl.ds(0, _CB)], buf.at[k], sem).wait()

        def write(chunk, buf):
            off = base + chunk * _CB
            for k in range(4):
                pltpu.sync_copy(buf.at[k], g_hbm.at[k, pl.ds(off, _CB)])

        start(0, bufa, sema)

        def body(t, carry):
            i = t * 2
            wait4(bufa, sema)
            start(i + 1, bufb, semb)
            write(i, bufa)
            wait4(bufb, semb)
            start(i + 2, bufa, sema)
            write(i + 1, bufb)
            return carry

        lax.fori_loop(0, NCH // 2 - 1, body, 0)
        wait4(bufa, sema)
        start(NCH - 1, bufb, semb)
        write(NCH - 2, bufa)
        wait4(bufb, semb)
        write(NCH - 1, bufb)

    return gather_kernel(xt, idx3)


def _tc_conv(fe, g, wt, b2, seg, es):
    _, C, E = fe.shape
    c_out = wt.shape[1]
    nb = es // _EB
    off = seg * nb

    def body(fe_ref, g_ref, w_ref, b_ref, y_ref, st_ref, sacc, qacc):
        i = pl.program_id(0)

        @pl.when(i == 0)
        def _():
            sacc[...] = jnp.zeros_like(sacc)
            qacc[...] = jnp.zeros_like(qacc)

        xb = fe_ref[0]
        w = w_ref[...]
        gg = g_ref[...]
        s13 = gg[0] + gg[2]
        s24 = gg[1] + gg[3]
        d13 = jnp.abs(gg[0] - gg[2])
        d24 = jnp.abs(gg[1] - gg[3])
        dn = (((1,), (1,)), ((), ()))
        acc = jnp.dot(w[0], xb, preferred_element_type=jnp.float32)
        acc += lax.dot_general(w[1], s13, dn, preferred_element_type=jnp.float32)
        acc += lax.dot_general(w[2], s24, dn, preferred_element_type=jnp.float32)
        acc += lax.dot_general(w[3], d13, dn, preferred_element_type=jnp.float32)
        acc += lax.dot_general(w[4], d24, dn, preferred_element_type=jnp.float32)
        y = jnp.maximum(acc + b_ref[...], 0.0)
        y_ref[...] = y
        sacc[...] += jnp.sum(y, axis=1, keepdims=True)
        qacc[...] += jnp.sum(y * y, axis=1, keepdims=True)

        @pl.when(i == nb - 1)
        def _():
            st_ref[...] = jnp.concatenate([sacc[...], qacc[...]], axis=1)

    return pl.pallas_call(
        body,
        grid=(nb,),
        in_specs=[
            pl.BlockSpec((1, C, _EB), lambda i: (0, 0, i + off)),
            pl.BlockSpec((4, _EB, C), lambda i: (0, i, 0)),
            pl.BlockSpec((5, c_out, C), lambda i: (0, 0, 0)),
            pl.BlockSpec((c_out, 1), lambda i: (0, 0)),
        ],
        out_specs=[
            pl.BlockSpec((c_out, _EB), lambda i: (0, i)),
            pl.BlockSpec((c_out, 2), lambda i: (0, 0)),
        ],
        out_shape=[
            jax.ShapeDtypeStruct((c_out, es), jnp.float32),
            jax.ShapeDtypeStruct((c_out, 2), jnp.float32),
        ],
        scratch_shapes=[
            pltpu.VMEM((c_out, 1), jnp.float32),
            pltpu.VMEM((c_out, 1), jnp.float32),
        ],
        compiler_params=pltpu.CompilerParams(
            dimension_semantics=("arbitrary",)),
    )(fe, g, wt, b2)


def _tc_norm(y, st_all, n_total):
    c_out, es = y.shape
    nb = es // _EB
    nseg = st_all.shape[0]
    inv_e = float(1.0 / n_total)

    def body(y_ref, st_ref, o_ref):
        stv = jnp.sum(st_ref[...], axis=0)
        mu = stv[:, 0:1] * inv_e
        var = stv[:, 1:2] * inv_e - mu * mu
        r = lax.rsqrt(var + EPS)
        o_ref[...] = ((y_ref[...] - mu) * r)[None]

    return pl.pallas_call(
        body,
        grid=(nb,),
        in_specs=[
            pl.BlockSpec((c_out, _EB), lambda i: (0, i)),
            pl.BlockSpec((nseg, c_out, 2), lambda i: (0, 0, 0)),
        ],
        out_specs=pl.BlockSpec((1, c_out, _EB), lambda i: (0, 0, i)),
        out_shape=jax.ShapeDtypeStruct((1, c_out, es), jnp.float32),
        compiler_params=pltpu.CompilerParams(
            dimension_semantics=("arbitrary",)),
    )(y, st_all)


def kernel(fe, gemm_edges, W, b):
    _, C, E = fe.shape
    c_out = W.shape[0]
    es = E // _NSEG
    xt = jnp.transpose(fe[0])
    wt = jnp.transpose(W, (2, 0, 1))
    b2 = b.reshape(c_out, 1)
    ew = -(-es // (_NW * _CB)) * _CB
    if (ew // _CB) % 2:
        ew += _CB
    idx = gemm_edges[0]

    ys, sts = [], []
    for s in range(_NSEG):
        idx_s = idx[s * es:(s + 1) * es]
        if _NW * ew != es:
            idx_s = jnp.concatenate(
                [idx_s, jnp.zeros((_NW * ew - es, 4), dtype=idx.dtype)],
                axis=0)
        idx3 = jnp.transpose(idx_s.reshape(_NW, ew, 4), (0, 2, 1))
        g = _sc_gather(xt, idx3)
        y, st = _tc_conv(fe, g, wt, b2, s, es)
        ys.append(y)
        sts.append(st)

    st_all = jnp.stack(sts)
    outs = [_tc_norm(y, st_all, E) for y in ys]
    return jnp.concatenate(outs, axis=2)


# speedup vs baseline: 2.2342x; 1.4015x over previous
import functools

import jax
import jax.numpy as jnp
from jax import lax
from jax.experimental import pallas as pl
from jax.experimental.pallas import tpu as pltpu
from jax.experimental.pallas import tpu_sc as plsc

EPS = 1e-5

_NC = 2
_NS = 16
_NW = _NC * _NS

_CB = 40

_NSEG = 1
_EB = 1280


def _sc_gather(xt, idx3):
    E, C = xt.shape
    EW = idx3.shape[2]
    NCH = EW // _CB
    EP = _NW * EW

    mesh = plsc.VectorSubcoreMesh(core_axis_name="c", subcore_axis_name="s")

    @functools.partial(
        pl.kernel,
        out_type=jax.ShapeDtypeStruct((4, EP, C), jnp.float32),
        mesh=mesh,
        scratch_types=[
            [pltpu.VMEM((EW,), jnp.int32) for _ in range(4)],
            pltpu.VMEM((4, _CB, C), jnp.float32),
            pltpu.VMEM((4, _CB, C), jnp.float32),
            pltpu.SemaphoreType.DMA,
            pltpu.SemaphoreType.DMA,
        ],
    )
    def gather_kernel(xt_hbm, idx3_hbm, g_hbm, idx_v, bufa, bufb, sema, semb):
        wid = lax.axis_index("s") * _NC + lax.axis_index("c")
        base = wid * EW
        for k in range(4):
            pltpu.sync_copy(idx3_hbm.at[wid, k], idx_v[k])

        def start(chunk, buf, sem):
            off = chunk * _CB
            for k in range(4):
                pltpu.async_copy(
                    xt_hbm.at[idx_v[k].at[pl.ds(off, _CB)]], buf.at[k], sem)

        def wait4(buf, sem):
            for k in range(4):
                pltpu.make_async_copy(
                    xt_hbm.at[pl.ds(0, _CB)], buf.at[k], sem).wait()

        def write(chunk, buf):
            off = base + chunk * _CB
            for k in range(4):
                pltpu.sync_copy(buf.at[k], g_hbm.at[k, pl.ds(off, _CB)])

        start(0, bufa, sema)

        def body(t, carry):
            i = t * 2
            wait4(bufa, sema)
            start(i + 1, bufb, semb)
            write(i, bufa)
            wait4(bufb, semb)
            start(i + 2, bufa, sema)
            write(i + 1, bufb)
            return carry

        lax.fori_loop(0, (NCH - 1) // 2, body, 0)
        if NCH % 2:
            wait4(bufa, sema)
            write(NCH - 1, bufa)
        else:
            wait4(bufa, sema)
            start(NCH - 1, bufb, semb)
            write(NCH - 2, bufa)
            wait4(bufb, semb)
            write(NCH - 1, bufb)

    return gather_kernel(xt, idx3)


def _tc_self(fe, w0, b2):
    _, C, E = fe.shape
    c_out = w0.shape[0]
    nb = E // _EB

    def body(fe_ref, w_ref, b_ref, y0_ref):
        y0_ref[...] = jnp.dot(
            w_ref[...], fe_ref[0],
            preferred_element_type=jnp.float32) + b_ref[...]

    return pl.pallas_call(
        body,
        grid=(nb,),
        in_specs=[
            pl.BlockSpec((1, C, _EB), lambda i: (0, 0, i)),
            pl.BlockSpec((c_out, C), lambda i: (0, 0)),
            pl.BlockSpec((c_out, 1), lambda i: (0, 0)),
        ],
        out_specs=pl.BlockSpec((c_out, _EB), lambda i: (0, i)),
        out_shape=jax.ShapeDtypeStruct((c_out, E), jnp.float32),
        compiler_params=pltpu.CompilerParams(
            dimension_semantics=("arbitrary",)),
    )(fe, w0, b2)


def _tc_conv(y0, g, wn, seg, es):
    c_out, E = y0.shape
    C = wn.shape[2]
    nb = es // _EB
    off = seg * nb

    def body(y0_ref, g_ref, w_ref, y_ref, st_ref, sacc, qacc):
        i = pl.program_id(0)

        @pl.when(i == 0)
        def _():
            sacc[...] = jnp.zeros_like(sacc)
            qacc[...] = jnp.zeros_like(qacc)

        w = w_ref[...]
        gg = g_ref[...]
        s13 = gg[0] + gg[2]
        s24 = gg[1] + gg[3]
        d13 = jnp.abs(gg[0] - gg[2])
        d24 = jnp.abs(gg[1] - gg[3])
        dn = (((1,), (1,)), ((), ()))
        acc = y0_ref[...]
        acc += lax.dot_general(w[0], s13, dn, preferred_element_type=jnp.float32)
        acc += lax.dot_general(w[1], s24, dn, preferred_element_type=jnp.float32)
        acc += lax.dot_general(w[2], d13, dn, preferred_element_type=jnp.float32)
        acc += lax.dot_general(w[3], d24, dn, preferred_element_type=jnp.float32)
        y = jnp.maximum(acc, 0.0)
        y_ref[...] = y
        sacc[...] += jnp.sum(y, axis=1, keepdims=True)
        qacc[...] += jnp.sum(y * y, axis=1, keepdims=True)

        @pl.when(i == nb - 1)
        def _():
            st_ref[...] = jnp.concatenate([sacc[...], qacc[...]], axis=1)

    return pl.pallas_call(
        body,
        grid=(nb,),
        in_specs=[
            pl.BlockSpec((c_out, _EB), lambda i: (0, i + off)),
            pl.BlockSpec((4, _EB, C), lambda i: (0, i, 0)),
            pl.BlockSpec((4, c_out, C), lambda i: (0, 0, 0)),
        ],
        out_specs=[
            pl.BlockSpec((c_out, _EB), lambda i: (0, i)),
            pl.BlockSpec((c_out, 2), lambda i: (0, 0)),
        ],
        out_shape=[
            jax.ShapeDtypeStruct((c_out, es), jnp.float32),
            jax.ShapeDtypeStruct((c_out, 2), jnp.float32),
        ],
        scratch_shapes=[
            pltpu.VMEM((c_out, 1), jnp.float32),
            pltpu.VMEM((c_out, 1), jnp.float32),
        ],
        compiler_params=pltpu.CompilerParams(
            dimension_semantics=("arbitrary",)),
    )(y0, g, wn)


def _tc_norm(y, st_all, n_total):
    c_out, es = y.shape
    nb = es // _EB
    nseg = st_all.shape[0]
    inv_e = float(1.0 / n_total)

    def body(y_ref, st_ref, o_ref):
        stv = jnp.sum(st_ref[...], axis=0)
        mu = stv[:, 0:1] * inv_e
        var = stv[:, 1:2] * inv_e - mu * mu
        r = lax.rsqrt(var + EPS)
        o_ref[...] = ((y_ref[...] - mu) * r)[None]

    return pl.pallas_call(
        body,
        grid=(nb,),
        in_specs=[
            pl.BlockSpec((c_out, _EB), lambda i: (0, i)),
            pl.BlockSpec((nseg, c_out, 2), lambda i: (0, 0, 0)),
        ],
        out_specs=pl.BlockSpec((1, c_out, _EB), lambda i: (0, 0, i)),
        out_shape=jax.ShapeDtypeStruct((1, c_out, es), jnp.float32),
        compiler_params=pltpu.CompilerParams(
            dimension_semantics=("arbitrary",)),
    )(y, st_all)


def kernel(fe, gemm_edges, W, b):
    _, C, E = fe.shape
    c_out = W.shape[0]
    es = E // _NSEG
    xt = jnp.transpose(fe[0])
    wn = jnp.transpose(W[:, :, 1:], (2, 0, 1))
    w0 = W[:, :, 0]
    b2 = b.reshape(c_out, 1)
    ew = -(-es // (_NW * _CB)) * _CB
    idx = gemm_edges[0]

    ys, sts = [], []
    for s in range(_NSEG):
        idx_s = idx[s * es:(s + 1) * es]
        if _NW * ew != es:
            idx_s = jnp.concatenate(
                [idx_s, jnp.zeros((_NW * ew - es, 4), dtype=idx.dtype)],
                axis=0)
        idx3 = jnp.transpose(idx_s.reshape(_NW, ew, 4), (0, 2, 1))
        g = _sc_gather(xt, idx3)
        if s == 0:
            y0 = _tc_self(fe, w0, b2)
        y, st = _tc_conv(y0, g, wn, s, es)
        ys.append(y)
        sts.append(st)

    st_all = jnp.stack(sts)
    outs = [_tc_norm(y, st_all, E) for y in ys]
    return jnp.concatenate(outs, axis=2)


# speedup vs baseline: 2.3203x; 1.0385x over previous
import functools

import jax
import jax.numpy as jnp
from jax import lax
from jax.experimental import pallas as pl
from jax.experimental.pallas import tpu as pltpu
from jax.experimental.pallas import tpu_sc as plsc

EPS = 1e-5

_NC = 2
_NS = 16
_NW = _NC * _NS

_CB = 40

_NSEG = 1
_EB = 1280


def _sc_gather(xt, idx3):
    E, C = xt.shape
    EW = idx3.shape[2]
    NCH = EW // _CB
    EP = _NW * EW

    mesh = plsc.VectorSubcoreMesh(core_axis_name="c", subcore_axis_name="s")

    @functools.partial(
        pl.kernel,
        out_type=jax.ShapeDtypeStruct((4, EP, C), jnp.float32),
        mesh=mesh,
        scratch_types=[
            [pltpu.VMEM((EW,), jnp.int32) for _ in range(4)],
            pltpu.VMEM((4, _CB, C), jnp.float32),
            pltpu.VMEM((4, _CB, C), jnp.float32),
            pltpu.SemaphoreType.DMA,
            pltpu.SemaphoreType.DMA,
        ],
    )
    def gather_kernel(xt_hbm, idx3_hbm, g_hbm, idx_v, bufa, bufb, sema, semb):
        wid = lax.axis_index("s") * _NC + lax.axis_index("c")
        base = wid * EW
        for k in range(4):
            pltpu.sync_copy(idx3_hbm.at[wid, k], idx_v[k])

        def start(chunk, buf, sem):
            off = chunk * _CB
            for k in range(4):
                pltpu.async_copy(
                    xt_hbm.at[idx_v[k].at[pl.ds(off, _CB)]], buf.at[k], sem)

        def wait4(buf, sem):
            for k in range(4):
                pltpu.make_async_copy(
                    xt_hbm.at[pl.ds(0, _CB)], buf.at[k], sem).wait()

        def write(chunk, buf):
            off = base + chunk * _CB
            for k in range(4):
                pltpu.sync_copy(buf.at[k], g_hbm.at[k, pl.ds(off, _CB)])

        start(0, bufa, sema)

        def body(t, carry):
            i = t * 2
            wait4(bufa, sema)
            start(i + 1, bufb, semb)
            write(i, bufa)
            wait4(bufb, semb)
            start(i + 2, bufa, sema)
            write(i + 1, bufb)
            return carry

        lax.fori_loop(0, (NCH - 1) // 2, body, 0)
        if NCH % 2:
            wait4(bufa, sema)
            write(NCH - 1, bufa)
        else:
            wait4(bufa, sema)
            start(NCH - 1, bufb, semb)
            write(NCH - 2, bufa)
            wait4(bufb, semb)
            write(NCH - 1, bufb)

    return gather_kernel(xt, idx3)


def _tc_conv(fe, g, wt, b2, seg, es):
    _, C, E = fe.shape
    c_out = wt.shape[1]
    nb = es // _EB
    off = seg * nb

    def body(fe_ref, g_ref, w_ref, b_ref, y_ref, st_ref, sacc, qacc):
        i = pl.program_id(0)

        @pl.when(i == 0)
        def _():
            sacc[...] = jnp.zeros_like(sacc)
            qacc[...] = jnp.zeros_like(qacc)

        xb = fe_ref[0]
        w = w_ref[...]
        gg = g_ref[...]
        s13 = gg[0] + gg[2]
        s24 = gg[1] + gg[3]
        d13 = jnp.abs(gg[0] - gg[2])
        d24 = jnp.abs(gg[1] - gg[3])
        dn = (((1,), (1,)), ((), ()))
        acc = jnp.dot(w[0], xb, preferred_element_type=jnp.float32)
        acc += lax.dot_general(w[1], s13, dn, preferred_element_type=jnp.float32)
        acc += lax.dot_general(w[2], s24, dn, preferred_element_type=jnp.float32)
        acc += lax.dot_general(w[3], d13, dn, preferred_element_type=jnp.float32)
        acc += lax.dot_general(w[4], d24, dn, preferred_element_type=jnp.float32)
        y = jnp.maximum(acc + b_ref[...], 0.0)
        y_ref[...] = y
        sacc[...] += jnp.sum(y, axis=1, keepdims=True)
        qacc[...] += jnp.sum(y * y, axis=1, keepdims=True)

        @pl.when(i == nb - 1)
        def _():
            st_ref[...] = jnp.concatenate([sacc[...], qacc[...]], axis=1)

    return pl.pallas_call(
        body,
        grid=(nb,),
        in_specs=[
            pl.BlockSpec((1, C, _EB), lambda i: (0, 0, i + off)),
            pl.BlockSpec((4, _EB, C), lambda i: (0, i, 0)),
            pl.BlockSpec((5, c_out, C), lambda i: (0, 0, 0)),
            pl.BlockSpec((c_out, 1), lambda i: (0, 0)),
        ],
        out_specs=[
            pl.BlockSpec((c_out, _EB), lambda i: (0, i)),
            pl.BlockSpec((c_out, 2), lambda i: (0, 0)),
        ],
        out_shape=[
            jax.ShapeDtypeStruct((c_out, es), jnp.float32),
            jax.ShapeDtypeStruct((c_out, 2), jnp.float32),
        ],
        scratch_shapes=[
            pltpu.VMEM((c_out, 1), jnp.float32),
            pltpu.VMEM((c_out, 1), jnp.float32),
        ],
        compiler_params=pltpu.CompilerParams(
            dimension_semantics=("arbitrary",)),
    )(fe, g, wt, b2)


def _tc_norm(y, st_all, n_total):
    c_out, es = y.shape
    nb = es // _EB
    nseg = st_all.shape[0]
    inv_e = float(1.0 / n_total)

    def body(y_ref, st_ref, o_ref):
        stv = jnp.sum(st_ref[...], axis=0)
        mu = stv[:, 0:1] * inv_e
        var = stv[:, 1:2] * inv_e - mu * mu
        r = lax.rsqrt(var + EPS)
        o_ref[...] = ((y_ref[...] - mu) * r)[None]

    return pl.pallas_call(
        body,
        grid=(nb,),
        in_specs=[
            pl.BlockSpec((c_out, _EB), lambda i: (0, i)),
            pl.BlockSpec((nseg, c_out, 2), lambda i: (0, 0, 0)),
        ],
        out_specs=pl.BlockSpec((1, c_out, _EB), lambda i: (0, 0, i)),
        out_shape=jax.ShapeDtypeStruct((1, c_out, es), jnp.float32),
        compiler_params=pltpu.CompilerParams(
            dimension_semantics=("arbitrary",)),
    )(y, st_all)


def kernel(fe, gemm_edges, W, b):
    _, C, E = fe.shape
    c_out = W.shape[0]
    es = E // _NSEG
    xt = jnp.transpose(fe[0])
    wt = jnp.transpose(W, (2, 0, 1))
    b2 = b.reshape(c_out, 1)
    ew = -(-es // (_NW * _CB)) * _CB
    idx = gemm_edges[0]

    ys, sts = [], []
    for s in range(_NSEG):
        idx_s = idx[s * es:(s + 1) * es]
        if _NW * ew != es:
            idx_s = jnp.concatenate(
                [idx_s, jnp.zeros((_NW * ew - es, 4), dtype=idx.dtype)],
                axis=0)
        idx3 = jnp.transpose(idx_s.reshape(_NW, ew, 4), (0, 2, 1))
        g = _sc_gather(xt, idx3)
        y, st = _tc_conv(fe, g, wt, b2, s, es)
        ys.append(y)
        sts.append(st)

    st_all = jnp.stack(sts)
    outs = [_tc_norm(y, st_all, E) for y in ys]
    return jnp.concatenate(outs, axis=2)
